# baseline (speedup 1.0000x reference)
import functools

import jax
import jax.numpy as jnp
from jax import lax
from jax.experimental import pallas as pl
from jax.experimental.pallas import tpu as pltpu
from jax.experimental.pallas import tpu_sc as plsc

B = 16384
NUM_REG = 4
NUM_CAT = 6
NUM_VAR = NUM_REG + NUM_CAT
VOCAB = 100000
D = 64

NC, NS, L = 2, 16, 16
NW = NC * NS
KPW = NUM_VAR * D // NW
TQ = 4096
NQ = B // TQ
GPQ = TQ // L

NEMB = 2 * NUM_CAT
NREGQ = 2 * NUM_REG * NQ


def _sc_body(aiT, tblT, w_hbm, b_hbm, outT, plane_v, chunk_v, idx_v, oh0,
             oh1, w_v, b_v, sem_o0, sem_o1, sem_s):
    wid = lax.axis_index("s") * NC + lax.axis_index("c")
    iota = lax.iota(jnp.int32, L)

    pltpu.sync_copy(w_hbm, w_v)
    pltpu.sync_copy(b_hbm, b_v)

    ohs = [oh0, oh1]
    sems = [sem_o0, sem_o1]
    last = [None, None]
    oh_turn = [0]

    def do_reg_quarter(kr, q):
        i = kr // 2
        d = 32 * (kr % 2) + wid
        dvec = iota * 0 + d
        ivec = iota * 0 + i
        wsp = plsc.load_gather(w_v, [ivec, dvec])
        bsp = plsc.load_gather(b_v, [ivec, dvec])
        p = oh_turn[0]
        oh_turn[0] ^= 1
        pltpu.sync_copy(aiT.at[i, pl.ds(q * TQ, TQ)], chunk_v)
        if last[p] is not None:
            last[p].wait()

        def _fma_body(g, _):
            x = chunk_v[pl.ds(g * L, L)]
            ohs[p][pl.ds(g * L, L)] = x * wsp + bsp
            return 0

        lax.fori_loop(0, GPQ, _fma_body, 0)
        last[p] = pltpu.async_copy(
            ohs[p], outT.at[i, d, pl.ds(q * TQ, TQ)], sems[p])

    reg_tasks = [(kr, q) for kr in range(2 * NUM_REG) for q in range(NQ)]
    fill = [[] for _ in range(NEMB)]
    for t, task in enumerate(reg_tasks):
        fill[t % NEMB].append(task)

    for e in range(NEMB):
        jc = e // 2
        d = 32 * (e % 2) + wid
        sdesc = pltpu.async_copy(tblT.at[jc, d], plane_v, sem_s)
        for (kr, q) in fill[e]:
            do_reg_quarter(kr, q)
        sdesc.wait()
        for q in range(NQ):
            p = oh_turn[0]
            oh_turn[0] ^= 1
            pltpu.sync_copy(aiT.at[NUM_REG + jc, pl.ds(q * TQ, TQ)],
                            chunk_v)

            if last[p] is not None:
                last[p].wait()

            def _g_body(g, _):
                iv = chunk_v[pl.ds(g * L, L)].astype(jnp.int32)
                ohs[p][pl.ds(g * L, L)] = plsc.load_gather(plane_v, [iv])
                return 0

            lax.fori_loop(0, GPQ, _g_body, 0, unroll=2)
            last[p] = pltpu.async_copy(
                ohs[p], outT.at[NUM_REG + jc, d, pl.ds(q * TQ, TQ)],
                sems[p])
    for p in range(2):
        if last[p] is not None:
            last[p].wait()


@jax.jit
def _run(aiT, tblT, W, b):
    mesh = plsc.VectorSubcoreMesh(core_axis_name="c", subcore_axis_name="s")
    f = functools.partial(
        pl.kernel,
        out_type=jax.ShapeDtypeStruct((NUM_VAR, D, B), jnp.float32),
        mesh=mesh,
        scratch_types=[
            pltpu.VMEM((VOCAB,), jnp.float32),
            pltpu.VMEM((TQ,), jnp.float32),
            pltpu.VMEM((TQ,), jnp.int32),
            pltpu.VMEM((TQ,), jnp.float32),
            pltpu.VMEM((TQ,), jnp.float32),
            pltpu.VMEM((NUM_REG, D), jnp.float32),
            pltpu.VMEM((NUM_REG, D), jnp.float32),
            pltpu.SemaphoreType.DMA,
            pltpu.SemaphoreType.DMA,
            pltpu.SemaphoreType.DMA,
        ],
        compiler_params=pltpu.CompilerParams(use_tc_tiling_on_sc=True,
                                             needs_layout_passes=False),
    )(_sc_body)
    return f(aiT, tblT, W, b)


def kernel(all_inputs, tables, W, b):
    aiT = all_inputs.T
    tblT = jnp.transpose(tables, (0, 2, 1))
    outT = _run(aiT, tblT, W, b)
    return jnp.transpose(outT, (2, 0, 1))


# speedup vs baseline: 1.6442x; 1.6442x over previous
import functools

import jax
import jax.numpy as jnp
from jax import lax
from jax.experimental import pallas as pl
from jax.experimental.pallas import tpu as pltpu
from jax.experimental.pallas import tpu_sc as plsc

B = 16384
NUM_REG = 4
NUM_CAT = 6
NUM_VAR = NUM_REG + NUM_CAT
VOCAB = 100000
D = 64

NC, NS, L = 2, 16, 16
NW = NC * NS
KPW = NUM_VAR * D // NW
TQ = 4096
NQ = B // TQ
GPQ = TQ // L

NEMB = 2 * NUM_CAT
NREGQ = 2 * NUM_REG * NQ


def _sc_body(aiT, tblT, w_hbm, b_hbm, outT, plane_v, chunk_v, chunk1_v, oh0,
             oh1, w_v, b_v, sem_o0, sem_o1, sem_s, sem_c0, sem_c1):
    wid = lax.axis_index("s") * NC + lax.axis_index("c")
    iota = lax.iota(jnp.int32, L)

    pltpu.sync_copy(w_hbm, w_v)
    pltpu.sync_copy(b_hbm, b_v)

    ohs = [oh0, oh1]
    sems = [sem_o0, sem_o1]
    last = [None, None]
    oh_turn = [0]

    def do_reg_quarter(kr, q):
        i = kr // 2
        d = 32 * (kr % 2) + wid
        dvec = iota * 0 + d
        ivec = iota * 0 + i
        wsp = plsc.load_gather(w_v, [ivec, dvec])
        bsp = plsc.load_gather(b_v, [ivec, dvec])
        p = oh_turn[0]
        oh_turn[0] ^= 1
        pltpu.sync_copy(aiT.at[i, pl.ds(q * TQ, TQ)], chunk_v)
        if last[p] is not None:
            last[p].wait()

        def _fma_body(g, _):
            x = chunk_v[pl.ds(g * L, L)]
            ohs[p][pl.ds(g * L, L)] = x * wsp + bsp
            return 0

        lax.fori_loop(0, GPQ, _fma_body, 0)
        last[p] = pltpu.async_copy(
            ohs[p], outT.at[i, d, pl.ds(q * TQ, TQ)], sems[p])

    reg_tasks = [(kr, q) for kr in range(2 * NUM_REG) for q in range(NQ)]
    fill = [[] for _ in range(NEMB)]
    for t, task in enumerate(reg_tasks):
        fill[t % NEMB].append(task)

    for e in range(NEMB):
        jc = e // 2
        d = 32 * (e % 2) + wid
        sdesc = pltpu.async_copy(tblT.at[jc, d], plane_v, sem_s)
        for (kr, q) in fill[e]:
            do_reg_quarter(kr, q)
        chunks = [chunk_v, chunk1_v]
        csems = [sem_c0, sem_c1]
        cds = [None, None]
        cds[0] = pltpu.async_copy(aiT.at[NUM_REG + jc, pl.ds(0, TQ)],
                                  chunks[0], csems[0])
        sdesc.wait()
        for q in range(NQ):
            p = oh_turn[0]
            oh_turn[0] ^= 1
            cq = q % 2
            if q + 1 < NQ:
                cds[cq ^ 1] = pltpu.async_copy(
                    aiT.at[NUM_REG + jc, pl.ds((q + 1) * TQ, TQ)],
                    chunks[cq ^ 1], csems[cq ^ 1])
            cds[cq].wait()

            if last[p] is not None:
                last[p].wait()

            def _g_body(g, _):
                iv = chunks[cq][pl.ds(g * L, L)].astype(jnp.int32)
                ohs[p][pl.ds(g * L, L)] = plsc.load_gather(plane_v, [iv])
                return 0

            lax.fori_loop(0, GPQ, _g_body, 0)
            last[p] = pltpu.async_copy(
                ohs[p], outT.at[NUM_REG + jc, d, pl.ds(q * TQ, TQ)],
                sems[p])
    for p in range(2):
        if last[p] is not None:
            last[p].wait()


@jax.jit
def _run(aiT, tblT, W, b):
    mesh = plsc.VectorSubcoreMesh(core_axis_name="c", subcore_axis_name="s")
    f = functools.partial(
        pl.kernel,
        out_type=jax.ShapeDtypeStruct((NUM_VAR, D, B), jnp.float32),
        mesh=mesh,
        scratch_types=[
            pltpu.VMEM((VOCAB,), jnp.float32),
            pltpu.VMEM((TQ,), jnp.float32),
            pltpu.VMEM((TQ,), jnp.float32),
            pltpu.VMEM((TQ,), jnp.float32),
            pltpu.VMEM((TQ,), jnp.float32),
            pltpu.VMEM((NUM_REG, D), jnp.float32),
            pltpu.VMEM((NUM_REG, D), jnp.float32),
            pltpu.SemaphoreType.DMA,
            pltpu.SemaphoreType.DMA,
            pltpu.SemaphoreType.DMA,
            pltpu.SemaphoreType.DMA,
            pltpu.SemaphoreType.DMA,
        ],
        compiler_params=pltpu.CompilerParams(use_tc_tiling_on_sc=True,
                                             needs_layout_passes=False),
    )(_sc_body)
    return f(aiT, tblT, W, b)


def kernel(all_inputs, tables, W, b):
    aiT = all_inputs.T
    tblT = jnp.transpose(tables, (0, 2, 1))
    outT = _run(aiT, tblT, W, b)
    return jnp.transpose(outT, (2, 0, 1))


# speedup vs baseline: 1.7532x; 1.0663x over previous
import functools

import jax
import jax.numpy as jnp
from jax import lax
from jax.experimental import pallas as pl
from jax.experimental.pallas import tpu as pltpu
from jax.experimental.pallas import tpu_sc as plsc

B = 16384
NUM_REG = 4
NUM_CAT = 6
NUM_VAR = NUM_REG + NUM_CAT
VOCAB = 100000
D = 64

NC, NS, L = 2, 16, 16
NW = NC * NS
KPW = NUM_VAR * D // NW
TQ = 4096
NQ = B // TQ
GPQ = TQ // L

NEMB = 2 * NUM_CAT
NREGQ = 2 * NUM_REG * NQ


def _sc_body(aiT, tblT, w_hbm, b_hbm, outT, plane_v, chunk_v, chunk1_v, oh0,
             oh1, w_v, b_v, sem_o0, sem_o1, sem_s, sem_c0, sem_c1):
    wid = lax.axis_index("s") * NC + lax.axis_index("c")
    iota = lax.iota(jnp.int32, L)

    pltpu.sync_copy(w_hbm, w_v)
    pltpu.sync_copy(b_hbm, b_v)

    ohs = [oh0, oh1]
    sems = [sem_o0, sem_o1]
    chunks = [chunk_v, chunk1_v]
    csems = [sem_c0, sem_c1]
    last = [None, None]
    oh_turn = [0]

    def fire_reg_chunk(kr, q, b):
        return pltpu.async_copy(
            aiT.at[kr // 2, pl.ds(q * TQ, TQ)], chunks[b], csems[b])

    def do_reg_quarter(kr, q, b):
        i = kr // 2
        d = 32 * (kr % 2) + wid
        dvec = iota * 0 + d
        ivec = iota * 0 + i
        wsp = plsc.load_gather(w_v, [ivec, dvec])
        bsp = plsc.load_gather(b_v, [ivec, dvec])
        p = oh_turn[0]
        oh_turn[0] ^= 1
        if last[p] is not None:
            last[p].wait()

        def _fma_body(g, _):
            x = chunks[b][pl.ds(g * L, L)]
            ohs[p][pl.ds(g * L, L)] = x * wsp + bsp
            return 0

        lax.fori_loop(0, GPQ, _fma_body, 0)
        last[p] = pltpu.async_copy(
            ohs[p], outT.at[i, d, pl.ds(q * TQ, TQ)], sems[p])

    reg_tasks = [(kr, q) for kr in range(2 * NUM_REG) for q in range(NQ)]
    fill = [[] for _ in range(NEMB)]
    for t, task in enumerate(reg_tasks):
        fill[t % NEMB].append(task)

    for e in range(NEMB):
        jc = e // 2
        d = 32 * (e % 2) + wid
        sdesc = pltpu.async_copy(tblT.at[jc, d], plane_v, sem_s)
        fl = fill[e]
        fd = [None, None]
        if fl:
            fd[0] = fire_reg_chunk(*fl[0], 0)
        for n, (kr, q) in enumerate(fl):
            b = n % 2
            if n + 1 < len(fl):
                fd[b ^ 1] = fire_reg_chunk(*fl[n + 1], b ^ 1)
            fd[b].wait()
            do_reg_quarter(kr, q, b)
        cds = [None, None]
        cds[0] = pltpu.async_copy(aiT.at[NUM_REG + jc, pl.ds(0, TQ)],
                                  chunks[0], csems[0])
        sdesc.wait()
        for q in range(NQ):
            p = oh_turn[0]
            oh_turn[0] ^= 1
            cq = q % 2
            if q + 1 < NQ:
                cds[cq ^ 1] = pltpu.async_copy(
                    aiT.at[NUM_REG + jc, pl.ds((q + 1) * TQ, TQ)],
                    chunks[cq ^ 1], csems[cq ^ 1])
            cds[cq].wait()

            if last[p] is not None:
                last[p].wait()

            def _g_body(g, _):
                iv = chunks[cq][pl.ds(g * L, L)].astype(jnp.int32)
                ohs[p][pl.ds(g * L, L)] = plsc.load_gather(plane_v, [iv])
                return 0

            lax.fori_loop(0, GPQ, _g_body, 0)
            last[p] = pltpu.async_copy(
                ohs[p], outT.at[NUM_REG + jc, d, pl.ds(q * TQ, TQ)],
                sems[p])
    for p in range(2):
        if last[p] is not None:
            last[p].wait()


@jax.jit
def _run(aiT, tblT, W, b):
    mesh = plsc.VectorSubcoreMesh(core_axis_name="c", subcore_axis_name="s")
    f = functools.partial(
        pl.kernel,
        out_type=jax.ShapeDtypeStruct((NUM_VAR, D, B), jnp.float32),
        mesh=mesh,
        scratch_types=[
            pltpu.VMEM((VOCAB,), jnp.float32),
            pltpu.VMEM((TQ,), jnp.float32),
            pltpu.VMEM((TQ,), jnp.float32),
            pltpu.VMEM((TQ,), jnp.float32),
            pltpu.VMEM((TQ,), jnp.float32),
            pltpu.VMEM((NUM_REG, D), jnp.float32),
            pltpu.VMEM((NUM_REG, D), jnp.float32),
            pltpu.SemaphoreType.DMA,
            pltpu.SemaphoreType.DMA,
            pltpu.SemaphoreType.DMA,
            pltpu.SemaphoreType.DMA,
            pltpu.SemaphoreType.DMA,
        ],
        compiler_params=pltpu.CompilerParams(use_tc_tiling_on_sc=True,
                                             needs_layout_passes=False),
    )(_sc_body)
    return f(aiT, tblT, W, b)


def kernel(all_inputs, tables, W, b):
    aiT = all_inputs.T
    tblT = jnp.transpose(tables, (0, 2, 1))
    outT = _run(aiT, tblT, W, b)
    return jnp.transpose(outT, (2, 0, 1))
